# Initial kernel scaffold; baseline (speedup 1.0000x reference)
import jax
import jax.numpy as jnp
from jax.experimental import pallas as pl


def kernel(x, table, W, b):
    raise NotImplementedError("write your pallas kernel here")


# speedup vs baseline: 2.0139x; 2.0139x over previous
import functools

import jax
import jax.numpy as jnp
from jax import lax
from jax.experimental import pallas as pl
from jax.experimental.pallas import tpu as pltpu
from jax.experimental.pallas import tpu_sc as plsc

VOCAB = 1000
EMBED = 128
B = 1024
T = 200
N_TOK = B * (T - 1)

NC, NS = 2, 16
NW = NC * NS
CHUNK = 128
CPW = 50
RPW = CHUNK * CPW
N_PAD = NW * RPW

MM_TILE = 1024
MM_GRID = N_TOK // MM_TILE


def _sc_gather_body(idx_hbm, table_hbm, emb_hbm, idx_v, rows_v, gsem):
    wid = lax.axis_index("s") * NC + lax.axis_index("c")
    base = wid * RPW
    pltpu.sync_copy(idx_hbm.at[wid], idx_v)

    def chunk(g, carry):
        pltpu.async_copy(table_hbm.at[idx_v.at[g]], rows_v, gsem).wait()
        pltpu.sync_copy(rows_v, emb_hbm.at[pl.ds(base + g * CHUNK, CHUNK)])
        return carry

    lax.fori_loop(0, CPW, chunk, 0)


_sc_gather = functools.partial(
    pl.kernel,
    out_type=jax.ShapeDtypeStruct((N_PAD, EMBED), jnp.float32),
    mesh=plsc.VectorSubcoreMesh(
        core_axis_name="c", subcore_axis_name="s", num_cores=NC, num_subcores=NS
    ),
    scratch_types=[
        pltpu.VMEM((CPW, CHUNK), jnp.int32),
        pltpu.VMEM((CHUNK, EMBED), jnp.float32),
        pltpu.SemaphoreType.DMA,
    ],
)(_sc_gather_body)


def _mm_body(emb_ref, w_ref, b_ref, out_ref):
    e = emb_ref[...].astype(jnp.bfloat16)
    acc = jnp.dot(e, w_ref[...], preferred_element_type=jnp.float32)
    out_ref[...] = acc + b_ref[...]


def _tc_matmul(emb, w_bf16, b2d):
    return pl.pallas_call(
        _mm_body,
        grid=(MM_GRID,),
        in_specs=[
            pl.BlockSpec((MM_TILE, EMBED), lambda i: (i, 0)),
            pl.BlockSpec((EMBED, VOCAB), lambda i: (0, 0)),
            pl.BlockSpec((1, VOCAB), lambda i: (0, 0)),
        ],
        out_specs=pl.BlockSpec((MM_TILE, VOCAB), lambda i: (i, 0)),
        out_shape=jax.ShapeDtypeStruct((N_TOK, VOCAB), jnp.float32),
    )(emb, w_bf16, b2d)


def kernel(x, table, W, b):
    x = x.astype(jnp.int32)
    idx = x[:, :-1] * VOCAB + x[:, 1:]
    idx_flat = idx.reshape(-1)
    idx_pad = jnp.concatenate(
        [idx_flat, jnp.zeros((N_PAD - N_TOK,), jnp.int32)]
    ).reshape(NW, CPW, CHUNK)
    emb = _sc_gather(idx_pad, table)
    logits = _tc_matmul(emb, W.astype(jnp.bfloat16), b.reshape(1, VOCAB))
    return logits.reshape(B, T - 1, VOCAB)


# speedup vs baseline: 2.0152x; 1.0006x over previous
import functools

import jax
import jax.numpy as jnp
from jax import lax
from jax.experimental import pallas as pl
from jax.experimental.pallas import tpu as pltpu
from jax.experimental.pallas import tpu_sc as plsc

VOCAB = 1000
EMBED = 128
B = 1024
T = 200
N_TOK = B * (T - 1)

NC, NS = 2, 16
NW = NC * NS
CHUNK = 128
CPW = 50
RPW = CHUNK * CPW
N_PAD = NW * RPW

MM_TILE = 1024
MM_GRID = N_TOK // MM_TILE


def _sc_gather_body(idx_hbm, table_hbm, emb_hbm, idx_v, rows_v, gsem):
    wid = lax.axis_index("s") * NC + lax.axis_index("c")
    base = wid * RPW
    pltpu.sync_copy(idx_hbm.at[wid], idx_v)

    def chunk(g, carry):
        pltpu.async_copy(table_hbm.at[idx_v.at[g]], rows_v, gsem).wait()
        pltpu.sync_copy(rows_v, emb_hbm.at[pl.ds(base + g * CHUNK, CHUNK)])
        return carry

    lax.fori_loop(0, CPW, chunk, 0)


_sc_gather = functools.partial(
    pl.kernel,
    out_type=jax.ShapeDtypeStruct((N_PAD, EMBED), jnp.float32),
    mesh=plsc.VectorSubcoreMesh(
        core_axis_name="c", subcore_axis_name="s", num_cores=NC, num_subcores=NS
    ),
    scratch_types=[
        pltpu.VMEM((CPW, CHUNK), jnp.int32),
        pltpu.VMEM((CHUNK, EMBED), jnp.float32),
        pltpu.SemaphoreType.DMA,
    ],
    compiler_params=pltpu.CompilerParams(use_tc_tiling_on_sc=True),
)(_sc_gather_body)


def _mm_body(emb_ref, w_ref, b_ref, out_ref):
    e = emb_ref[...].astype(jnp.bfloat16)
    acc = jnp.dot(e, w_ref[...], preferred_element_type=jnp.float32)
    out_ref[...] = acc + b_ref[...]


def _tc_matmul(emb, w_bf16, b2d):
    return pl.pallas_call(
        _mm_body,
        grid=(MM_GRID,),
        in_specs=[
            pl.BlockSpec((MM_TILE, EMBED), lambda i: (i, 0)),
            pl.BlockSpec((EMBED, VOCAB), lambda i: (0, 0)),
            pl.BlockSpec((1, VOCAB), lambda i: (0, 0)),
        ],
        out_specs=pl.BlockSpec((MM_TILE, VOCAB), lambda i: (i, 0)),
        out_shape=jax.ShapeDtypeStruct((N_TOK, VOCAB), jnp.float32),
    )(emb, w_bf16, b2d)


def kernel(x, table, W, b):
    x = x.astype(jnp.int32)
    idx = x[:, :-1] * VOCAB + x[:, 1:]
    idx_flat = idx.reshape(-1)
    idx_pad = jnp.concatenate(
        [idx_flat, jnp.zeros((N_PAD - N_TOK,), jnp.int32)]
    ).reshape(NW, CPW, CHUNK)
    emb = _sc_gather(idx_pad, table)
    logits = _tc_matmul(emb, W.astype(jnp.bfloat16), b.reshape(1, VOCAB))
    return logits.reshape(B, T - 1, VOCAB)


# speedup vs baseline: 3.1635x; 1.5698x over previous
import functools

import jax
import jax.numpy as jnp
from jax import lax
from jax.experimental import pallas as pl
from jax.experimental.pallas import tpu as pltpu
from jax.experimental.pallas import tpu_sc as plsc

VOCAB = 1000
EMBED = 128
B = 1024
T = 200
TOUT = T - 1
TPAD = 200

NC, NS = 2, 16
NW = NC * NS
N_PAD = B * TPAD
RPW = N_PAD // NW
CHUNK = 128
CPW = RPW // CHUNK

BB = 8
MM_GRID = B // BB


def _sc_gather_body(idx_hbm, table_hbm, emb_hbm, idx_v, rows_v, gsem):
    wid = lax.axis_index("s") * NC + lax.axis_index("c")
    base = wid * RPW
    pltpu.sync_copy(idx_hbm.at[wid], idx_v)

    def chunk(g, carry):
        pltpu.async_copy(table_hbm.at[idx_v.at[g]], rows_v, gsem).wait()
        pltpu.sync_copy(rows_v, emb_hbm.at[pl.ds(base + g * CHUNK, CHUNK)])
        return carry

    lax.fori_loop(0, CPW, chunk, 0)


_sc_gather = functools.partial(
    pl.kernel,
    out_type=jax.ShapeDtypeStruct((N_PAD, EMBED), jnp.float32),
    mesh=plsc.VectorSubcoreMesh(
        core_axis_name="c", subcore_axis_name="s", num_cores=NC, num_subcores=NS
    ),
    scratch_types=[
        pltpu.VMEM((CPW, CHUNK), jnp.int32),
        pltpu.VMEM((CHUNK, EMBED), jnp.float32),
        pltpu.SemaphoreType.DMA,
    ],
)(_sc_gather_body)


def _mm_body(emb_ref, w_ref, b_ref, out_ref):
    e = emb_ref[...].reshape(BB * TPAD, EMBED).astype(jnp.bfloat16)
    acc = jnp.dot(e, w_ref[...], preferred_element_type=jnp.float32)
    acc = acc + b_ref[...]
    out_ref[...] = acc.reshape(BB, TPAD, VOCAB)[:, :TOUT, :]


def _tc_matmul(emb3, w_bf16, b2d):
    return pl.pallas_call(
        _mm_body,
        grid=(MM_GRID,),
        in_specs=[
            pl.BlockSpec((BB, TPAD, EMBED), lambda i: (i, 0, 0)),
            pl.BlockSpec((EMBED, VOCAB), lambda i: (0, 0)),
            pl.BlockSpec((1, VOCAB), lambda i: (0, 0)),
        ],
        out_specs=pl.BlockSpec((BB, TOUT, VOCAB), lambda i: (i, 0, 0)),
        out_shape=jax.ShapeDtypeStruct((B, TOUT, VOCAB), jnp.float32),
    )(emb3, w_bf16, b2d)


def kernel(x, table, W, b):
    x = x.astype(jnp.int32)
    idx = jnp.concatenate(
        [x[:, :-1] * VOCAB + x[:, 1:], jnp.zeros((B, 1), jnp.int32)], axis=1
    )
    idx_pad = idx.reshape(NW, CPW, CHUNK)
    emb = _sc_gather(idx_pad, table)
    emb3 = emb.reshape(B, TPAD, EMBED)
    return _tc_matmul(emb3, W.astype(jnp.bfloat16), b.reshape(1, VOCAB))


# speedup vs baseline: 3.1844x; 1.0066x over previous
import functools

import jax
import jax.numpy as jnp
from jax import lax
from jax.experimental import pallas as pl
from jax.experimental.pallas import tpu as pltpu
from jax.experimental.pallas import tpu_sc as plsc

VOCAB = 1000
EMBED = 128
B = 1024
T = 200
TOUT = T - 1
TPAD = 200

NC, NS = 2, 16
NW = NC * NS
N_PAD = B * TPAD
RPW = N_PAD // NW
CHUNK = 128
CPW = RPW // CHUNK

BB = 16
MM_GRID = B // BB


def _sc_gather_body(idx_hbm, table_hbm, emb_hbm, idx_v, rows_v, gsem):
    wid = lax.axis_index("s") * NC + lax.axis_index("c")
    base = wid * RPW
    pltpu.sync_copy(idx_hbm.at[wid], idx_v)

    def chunk(g, carry):
        pltpu.async_copy(table_hbm.at[idx_v.at[g]], rows_v, gsem).wait()
        pltpu.sync_copy(rows_v, emb_hbm.at[pl.ds(base + g * CHUNK, CHUNK)])
        return carry

    lax.fori_loop(0, CPW, chunk, 0)


_sc_gather = functools.partial(
    pl.kernel,
    out_type=jax.ShapeDtypeStruct((N_PAD, EMBED), jnp.float32),
    mesh=plsc.VectorSubcoreMesh(
        core_axis_name="c", subcore_axis_name="s", num_cores=NC, num_subcores=NS
    ),
    scratch_types=[
        pltpu.VMEM((CPW, CHUNK), jnp.int32),
        pltpu.VMEM((CHUNK, EMBED), jnp.float32),
        pltpu.SemaphoreType.DMA,
    ],
)(_sc_gather_body)


def _mm_body(emb_ref, w_ref, b_ref, out_ref):
    e = emb_ref[...].astype(jnp.bfloat16)
    acc = jnp.dot(e, w_ref[...], preferred_element_type=jnp.float32)
    acc = acc + b_ref[...]
    out_ref[...] = acc.reshape(BB, TPAD, VOCAB)[:, :TOUT, :]


def _tc_matmul(emb, w_bf16, b2d):
    return pl.pallas_call(
        _mm_body,
        grid=(MM_GRID,),
        in_specs=[
            pl.BlockSpec((BB * TPAD, EMBED), lambda i: (i, 0)),
            pl.BlockSpec((EMBED, VOCAB), lambda i: (0, 0)),
            pl.BlockSpec((1, VOCAB), lambda i: (0, 0)),
        ],
        out_specs=pl.BlockSpec((BB, TOUT, VOCAB), lambda i: (i, 0, 0)),
        out_shape=jax.ShapeDtypeStruct((B, TOUT, VOCAB), jnp.float32),
    )(emb, w_bf16, b2d)


def kernel(x, table, W, b):
    x = x.astype(jnp.int32)
    idx = jnp.concatenate(
        [x[:, :-1] * VOCAB + x[:, 1:], jnp.zeros((B, 1), jnp.int32)], axis=1
    )
    idx_pad = idx.reshape(NW, CPW, CHUNK)
    emb = _sc_gather(idx_pad, table)
    return _tc_matmul(emb, W.astype(jnp.bfloat16), b.reshape(1, VOCAB))


# speedup vs baseline: 3.2146x; 1.0095x over previous
import functools

import jax
import jax.numpy as jnp
from jax import lax
from jax.experimental import pallas as pl
from jax.experimental.pallas import tpu as pltpu
from jax.experimental.pallas import tpu_sc as plsc

VOCAB = 1000
EMBED = 128
B = 1024
T = 200
TOUT = T - 1
TPAD = 200

NC, NS = 2, 16
NW = NC * NS
CHUNK = 128

NSEG = 2
BPS = B // NSEG
SEG_ROWS = BPS * TPAD
CPW = SEG_ROWS // (NW * CHUNK)
RPW = CPW * CHUNK

BB = 16
MM_GRID = BPS // BB


def _sc_gather_body(idx_hbm, table_hbm, emb_hbm, idx_v, rows_v, gsem):
    wid = lax.axis_index("s") * NC + lax.axis_index("c")
    base = wid * RPW
    pltpu.sync_copy(idx_hbm.at[wid], idx_v)

    def chunk(g, carry):
        pltpu.async_copy(table_hbm.at[idx_v.at[g]], rows_v, gsem).wait()
        pltpu.sync_copy(rows_v, emb_hbm.at[pl.ds(base + g * CHUNK, CHUNK)])
        return carry

    lax.fori_loop(0, CPW, chunk, 0)


_sc_gather = functools.partial(
    pl.kernel,
    out_type=jax.ShapeDtypeStruct((SEG_ROWS, EMBED), jnp.float32),
    mesh=plsc.VectorSubcoreMesh(
        core_axis_name="c", subcore_axis_name="s", num_cores=NC, num_subcores=NS
    ),
    scratch_types=[
        pltpu.VMEM((CPW, CHUNK), jnp.int32),
        pltpu.VMEM((CHUNK, EMBED), jnp.float32),
        pltpu.SemaphoreType.DMA,
    ],
)(_sc_gather_body)


def _mm_body_first(emb_ref, w_ref, b_ref, out_ref):
    e = emb_ref[...].astype(jnp.bfloat16)
    acc = jnp.dot(e, w_ref[...], preferred_element_type=jnp.float32)
    acc = acc + b_ref[...]
    out_ref[...] = acc.reshape(BB, TPAD, VOCAB)[:, :TOUT, :]


def _mm_body_next(emb_ref, w_ref, b_ref, prev_ref, out_ref):
    del prev_ref
    _mm_body_first(emb_ref, w_ref, b_ref, out_ref)


def _tc_matmul(emb, w_bf16, b2d, seg, prev=None):
    off = seg * MM_GRID
    in_specs = [
        pl.BlockSpec((BB * TPAD, EMBED), lambda i: (i, 0)),
        pl.BlockSpec((EMBED, VOCAB), lambda i: (0, 0)),
        pl.BlockSpec((1, VOCAB), lambda i: (0, 0)),
    ]
    args = (emb, w_bf16, b2d)
    body = _mm_body_first
    aliases = {}
    if prev is not None:
        in_specs.append(pl.BlockSpec(memory_space=pl.ANY))
        args = args + (prev,)
        body = _mm_body_next
        aliases = {3: 0}
    return pl.pallas_call(
        body,
        grid=(MM_GRID,),
        in_specs=in_specs,
        out_specs=pl.BlockSpec((BB, TOUT, VOCAB), lambda i: (i + off, 0, 0)),
        out_shape=jax.ShapeDtypeStruct((B, TOUT, VOCAB), jnp.float32),
        input_output_aliases=aliases,
    )(*args)


def kernel(x, table, W, b):
    x = x.astype(jnp.int32)
    idx = jnp.concatenate(
        [x[:, :-1] * VOCAB + x[:, 1:], jnp.zeros((B, 1), jnp.int32)], axis=1
    )
    w_bf16 = W.astype(jnp.bfloat16)
    b2d = b.reshape(1, VOCAB)
    embs = [
        _sc_gather(
            idx[seg * BPS:(seg + 1) * BPS].reshape(NW, CPW, CHUNK), table
        )
        for seg in range(NSEG)
    ]
    logits = _tc_matmul(embs[0], w_bf16, b2d, 0)
    for seg in range(1, NSEG):
        logits = _tc_matmul(embs[seg], w_bf16, b2d, seg, prev=logits)
    return logits


# speedup vs baseline: 3.3106x; 1.0299x over previous
import functools

import jax
import jax.numpy as jnp
from jax import lax
from jax.experimental import pallas as pl
from jax.experimental.pallas import tpu as pltpu
from jax.experimental.pallas import tpu_sc as plsc

VOCAB = 1000
EMBED = 128
B = 1024
T = 200
TOUT = T - 1
TPAD = 200

NC, NS = 2, 16
NW = NC * NS
CHUNK = 128

NSEG = 2
BPS = B // NSEG
SEG_ROWS = BPS * TPAD
CPW = SEG_ROWS // (NW * CHUNK)
RPW = CPW * CHUNK

BB = 16
MM_GRID = BPS // BB


def _sc_gather_body(idx_hbm, table_hbm, emb_hbm, idx_v, rows_v, bf_v, gsem, wsem):
    wid = lax.axis_index("s") * NC + lax.axis_index("c")
    base = wid * RPW
    pltpu.sync_copy(idx_hbm.at[wid], idx_v)

    pltpu.async_copy(table_hbm.at[idx_v.at[0]], rows_v.at[0], gsem.at[0])

    def convert(src, dst):
        def row(r, c):
            for j in range(EMBED // 32):
                a = src[r, pl.ds(32 * j, 16)]
                bh = src[r, pl.ds(32 * j + 16, 16)]
                dst[r, pl.ds(32 * j, 32)] = plsc.pack(
                    a, bh, format=plsc.PackFormat.INTERLEAVED
                )
            return c

        lax.fori_loop(0, CHUNK, row, 0)

    def do_chunk(g, bbuf, last):
        pltpu.make_async_copy(
            table_hbm.at[idx_v.at[g]], rows_v.at[bbuf], gsem.at[bbuf]
        ).wait()

        if not last:

            @pl.when(g + 1 < CPW)
            def _():
                pltpu.async_copy(
                    table_hbm.at[idx_v.at[g + 1]],
                    rows_v.at[1 - bbuf],
                    gsem.at[1 - bbuf],
                )

        @pl.when(g >= 2)
        def _():
            pltpu.make_async_copy(
                bf_v.at[bbuf],
                emb_hbm.at[pl.ds(base, CHUNK)],
                wsem.at[bbuf],
            ).wait()

        convert(rows_v.at[bbuf], bf_v.at[bbuf])
        pltpu.async_copy(
            bf_v.at[bbuf],
            emb_hbm.at[pl.ds(base + g * CHUNK, CHUNK)],
            wsem.at[bbuf],
        )

    def outer(g0, carry):
        for bbuf in (0, 1):
            do_chunk(g0 * 2 + bbuf, bbuf, last=False)
        return carry

    lax.fori_loop(0, CPW // 2, outer, 0)
    if CPW % 2:
        do_chunk(jnp.int32(CPW - 1), (CPW - 1) % 2, last=True)

    for bbuf in (0, 1):
        pltpu.make_async_copy(
            bf_v.at[bbuf], emb_hbm.at[pl.ds(base, CHUNK)], wsem.at[bbuf]
        ).wait()


_sc_gather = functools.partial(
    pl.kernel,
    out_type=jax.ShapeDtypeStruct((SEG_ROWS, EMBED), jnp.bfloat16),
    mesh=plsc.VectorSubcoreMesh(
        core_axis_name="c", subcore_axis_name="s", num_cores=NC, num_subcores=NS
    ),
    scratch_types=[
        pltpu.VMEM((CPW, CHUNK), jnp.int32),
        pltpu.VMEM((2, CHUNK, EMBED), jnp.float32),
        pltpu.VMEM((2, CHUNK, EMBED), jnp.bfloat16),
        pltpu.SemaphoreType.DMA((2,)),
        pltpu.SemaphoreType.DMA((2,)),
    ],
    compiler_params=pltpu.CompilerParams(needs_layout_passes=False),
)(_sc_gather_body)


def _mm_body_first(emb_ref, w_ref, b_ref, out_ref):
    acc = jnp.dot(emb_ref[...], w_ref[...], preferred_element_type=jnp.float32)
    acc = acc + b_ref[...]
    out_ref[...] = acc.reshape(BB, TPAD, VOCAB)[:, :TOUT, :]


def _mm_body_next(emb_ref, w_ref, b_ref, prev_ref, out_ref):
    del prev_ref
    _mm_body_first(emb_ref, w_ref, b_ref, out_ref)


def _tc_matmul(emb, w_bf16, b2d, seg, prev=None):
    off = seg * MM_GRID
    in_specs = [
        pl.BlockSpec((BB * TPAD, EMBED), lambda i: (i, 0)),
        pl.BlockSpec((EMBED, VOCAB), lambda i: (0, 0)),
        pl.BlockSpec((1, VOCAB), lambda i: (0, 0)),
    ]
    args = (emb, w_bf16, b2d)
    body = _mm_body_first
    aliases = {}
    if prev is not None:
        in_specs.append(pl.BlockSpec(memory_space=pl.ANY))
        args = args + (prev,)
        body = _mm_body_next
        aliases = {3: 0}
    return pl.pallas_call(
        body,
        grid=(MM_GRID,),
        in_specs=in_specs,
        out_specs=pl.BlockSpec((BB, TOUT, VOCAB), lambda i: (i + off, 0, 0)),
        out_shape=jax.ShapeDtypeStruct((B, TOUT, VOCAB), jnp.float32),
        input_output_aliases=aliases,
    )(*args)


def _w_perm():
    perm = []
    for j in range(EMBED // 32):
        for i in range(16):
            perm.append(32 * j + i)
            perm.append(32 * j + 16 + i)
    return jnp.array(perm, dtype=jnp.int32)


def kernel(x, table, W, b):
    x = x.astype(jnp.int32)
    idx = jnp.concatenate(
        [x[:, :-1] * VOCAB + x[:, 1:], jnp.zeros((B, 1), jnp.int32)], axis=1
    )
    w_bf16 = W[_w_perm(), :].astype(jnp.bfloat16)
    b2d = b.reshape(1, VOCAB)
    embs = [
        _sc_gather(
            idx[seg * BPS:(seg + 1) * BPS].reshape(NW, CPW, CHUNK), table
        )
        for seg in range(NSEG)
    ]
    logits = _tc_matmul(embs[0], w_bf16, b2d, 0)
    for seg in range(1, NSEG):
        logits = _tc_matmul(embs[seg], w_bf16, b2d, seg, prev=logits)
    return logits
